# Initial kernel scaffold; baseline (speedup 1.0000x reference)
import jax
import jax.numpy as jnp
from jax.experimental import pallas as pl


def kernel(action, table):
    raise NotImplementedError("write your pallas kernel here")


# speedup vs baseline: 10.6766x; 10.6766x over previous
import jax
import jax.numpy as jnp
from jax import lax
from jax.experimental import pallas as pl
from jax.experimental.pallas import tpu as pltpu
from jax.experimental.pallas import tpu_sc as plsc

NUM_ACTIONS = 100000
ACTION_DIM = 32
BATCH = 16384
HIST = 50

NC, NS = 2, 16
NW = NC * NS
B_TOTAL = BATCH * HIST
PER_W = B_TOTAL // NW
CHUNK = 128
K = PER_W // CHUNK
SUB = 8
MEGA = CHUNK * SUB
N_MEGA = K // SUB


def _gather_body(idx_hbm, table_hbm, out_hbm, idx_v, rows_v, gsem):
    cid = lax.axis_index("c")
    sid = lax.axis_index("s")
    wid = sid * NC + cid
    base = wid * PER_W

    pltpu.sync_copy(idx_hbm.at[wid], idx_v)

    @pl.loop(0, N_MEGA)
    def _mega(m):
        descs = []
        for j in range(SUB):
            descs.append(pltpu.async_copy(
                table_hbm.at[idx_v.at[m * SUB + j]],
                rows_v.at[pl.ds(j * CHUNK, CHUNK)],
                gsem,
            ))
        for d in descs:
            d.wait()
        pltpu.sync_copy(rows_v, out_hbm.at[pl.ds(base + m * MEGA, MEGA)])


_gather = pl.kernel(
    _gather_body,
    out_type=jax.ShapeDtypeStruct((B_TOTAL, ACTION_DIM), jnp.float32),
    mesh=plsc.VectorSubcoreMesh(core_axis_name="c", subcore_axis_name="s"),
    scratch_types=[
        pltpu.VMEM((K, CHUNK), jnp.int32),
        pltpu.VMEM((MEGA, ACTION_DIM), jnp.float32),
        pltpu.SemaphoreType.DMA,
    ],
    compiler_params=pltpu.CompilerParams(use_tc_tiling_on_sc=False),
)


def kernel(action, table):
    idx = action.reshape(NW, K, CHUNK).astype(jnp.int32)
    out = _gather(idx, table)
    return out.reshape(BATCH, HIST * ACTION_DIM)


# speedup vs baseline: 11.1977x; 1.0488x over previous
import jax
import jax.numpy as jnp
from jax import lax
from jax.experimental import pallas as pl
from jax.experimental.pallas import tpu as pltpu
from jax.experimental.pallas import tpu_sc as plsc

NUM_ACTIONS = 100000
ACTION_DIM = 32
BATCH = 16384
HIST = 50

NC, NS = 2, 16
NW = NC * NS
B_TOTAL = BATCH * HIST
PER_W = B_TOTAL // NW
CHUNK = 128
K = PER_W // CHUNK
SUB = 8
MEGA = CHUNK * SUB
N_MEGA = K // SUB


def _issue(table_hbm, idx_v, rows, sem, m):
    for j in range(SUB):
        pltpu.async_copy(
            table_hbm.at[idx_v.at[m * SUB + j]],
            rows.at[pl.ds(j * CHUNK, CHUNK)],
            sem,
        )


def _drain(out_hbm, rows, sem):
    pltpu.make_async_copy(out_hbm.at[pl.ds(0, MEGA)], rows, sem).wait()


def _gather_body(idx_hbm, table_hbm, out_hbm, idx_v, rows0, rows1, sem0, sem1):
    cid = lax.axis_index("c")
    sid = lax.axis_index("s")
    wid = sid * NC + cid
    base = wid * PER_W

    pltpu.sync_copy(idx_hbm.at[wid], idx_v)

    bufs = (rows0, rows1)
    sems = (sem0, sem1)
    _issue(table_hbm, idx_v, bufs[0], sems[0], 0)

    @pl.loop(0, (N_MEGA - 1) // 2)
    def _pair(g):
        for b in range(2):
            m = 2 * g + b
            _issue(table_hbm, idx_v, bufs[1 - b], sems[1 - b], m + 1)
            _drain(out_hbm, bufs[b], sems[b])
            pltpu.sync_copy(bufs[b], out_hbm.at[pl.ds(base + m * MEGA, MEGA)])

    last = N_MEGA - 1
    _drain(out_hbm, bufs[last % 2], sems[last % 2])
    pltpu.sync_copy(bufs[last % 2], out_hbm.at[pl.ds(base + last * MEGA, MEGA)])


_gather = pl.kernel(
    _gather_body,
    out_type=jax.ShapeDtypeStruct((B_TOTAL, ACTION_DIM), jnp.float32),
    mesh=plsc.VectorSubcoreMesh(core_axis_name="c", subcore_axis_name="s"),
    scratch_types=[
        pltpu.VMEM((K, CHUNK), jnp.int32),
        pltpu.VMEM((MEGA, ACTION_DIM), jnp.float32),
        pltpu.VMEM((MEGA, ACTION_DIM), jnp.float32),
        pltpu.SemaphoreType.DMA,
        pltpu.SemaphoreType.DMA,
    ],
    compiler_params=pltpu.CompilerParams(use_tc_tiling_on_sc=False),
)


def kernel(action, table):
    idx = action.reshape(NW, K, CHUNK).astype(jnp.int32)
    out = _gather(idx, table)
    return out.reshape(BATCH, HIST * ACTION_DIM)
